# Initial kernel scaffold; baseline (speedup 1.0000x reference)
import jax
import jax.numpy as jnp
from jax.experimental import pallas as pl


def kernel(sentences1, sentences2, emb_table):
    raise NotImplementedError("write your pallas kernel here")


# speedup vs baseline: 1.2966x; 1.2966x over previous
import functools

import jax
import jax.numpy as jnp
from jax import lax
from jax.experimental import pallas as pl
from jax.experimental.pallas import tpu as pltpu
from jax.experimental.pallas import tpu_sc as plsc

VOCAB = 1000000
EMB = 32
B = 4096
L = 50
TEMP = 0.05

NC, NS = 2, 16
NW = NC * NS
SENTS = 2 * B
S_PER_W = SENTS // NW
IDX_PER_W = S_PER_W * L
IDX_COLS = 128
IDX_ROWS_PER_W = IDX_PER_W // IDX_COLS
CHUNK_S = 64
CHUNK_I = CHUNK_S * L
GROUPS = CHUNK_I // IDX_COLS
N_CHUNKS = S_PER_W // CHUNK_S


def _make_sc_pool():
    mesh = plsc.VectorSubcoreMesh(
        core_axis_name="c", subcore_axis_name="s", num_cores=NC, num_subcores=NS
    )

    @functools.partial(
        pl.kernel,
        out_type=jax.ShapeDtypeStruct((SENTS, EMB), jnp.float32),
        mesh=mesh,
        scratch_types=[
            pltpu.VMEM((IDX_ROWS_PER_W, IDX_COLS), jnp.int32),
            pltpu.VMEM((CHUNK_I, EMB), jnp.float32),
            pltpu.VMEM((S_PER_W, EMB), jnp.float32),
            pltpu.SemaphoreType.DMA,
        ],
        compiler_params=pltpu.CompilerParams(use_tc_tiling_on_sc=False),
    )
    def sc_pool(sents_hbm, table_hbm, out_hbm, idx_v, rows_v, acc_v, sem):
        wid = lax.axis_index("s") * NC + lax.axis_index("c")
        pltpu.sync_copy(sents_hbm.at[wid], idx_v)
        for k in range(N_CHUNKS):
            def fire(j, _):
                pltpu.async_copy(
                    table_hbm.at[idx_v.at[k * GROUPS + j]],
                    rows_v.at[pl.ds(j * IDX_COLS, IDX_COLS)],
                    sem,
                )
                return _

            lax.fori_loop(0, GROUPS, fire, 0)
            pltpu.make_async_copy(table_hbm.at[pl.ds(0, CHUNK_I)], rows_v, sem).wait()

            def per_sentence(s, _):
                def per_tok(j, carry):
                    a0, a1 = carry
                    r = s * L + j
                    return a0 + rows_v[r, 0:16], a1 + rows_v[r, 16:32]

                a0, a1 = lax.fori_loop(
                    0, L, per_tok,
                    (jnp.zeros(16, jnp.float32), jnp.zeros(16, jnp.float32)),
                )
                acc_v[k * CHUNK_S + s, 0:16] = a0
                acc_v[k * CHUNK_S + s, 16:32] = a1
                return _

            lax.fori_loop(0, CHUNK_S, per_sentence, 0)
        pltpu.sync_copy(acc_v, out_hbm.at[pl.ds(wid * S_PER_W, S_PER_W)])

    return sc_pool


_sc_pool = _make_sc_pool()

BLK = 256


def _tc_loss_body(p1_ref, p2_ref, s1_ref, s2_ref, row0_ref, out_ref):
    i = pl.program_id(0)
    row0 = row0_ref[...]

    cnt2 = jnp.sum((s2_ref[...] == 0).astype(jnp.float32), axis=1, keepdims=True)
    e2 = p2_ref[...] - cnt2 * row0
    n2 = e2 * lax.rsqrt(
        jnp.maximum(jnp.sum(e2 * e2, axis=1, keepdims=True), 1e-24)
    )

    cnt1 = jnp.sum((s1_ref[...] == 0).astype(jnp.float32), axis=1, keepdims=True)
    e1 = p1_ref[...] - cnt1 * row0
    n1 = e1 * lax.rsqrt(
        jnp.maximum(jnp.sum(e1 * e1, axis=1, keepdims=True), 1e-24)
    )

    logits = lax.dot_general(
        n1, n2, (((1,), (1,)), ((), ())),
        precision=lax.Precision.HIGHEST,
        preferred_element_type=jnp.float32,
    ) * (1.0 / TEMP)
    m = jnp.max(logits, axis=1, keepdims=True)
    logz = m[:, 0] + jnp.log(jnp.sum(jnp.exp(logits - m), axis=1))
    col = lax.broadcasted_iota(jnp.int32, logits.shape, 1)
    row = lax.broadcasted_iota(jnp.int32, logits.shape, 0) + i * BLK
    diag = jnp.sum(jnp.where(col == row, logits, 0.0), axis=1)
    part = jnp.sum(logz - diag).reshape(1, 1) * (1.0 / B)

    @pl.when(i == 0)
    def _():
        out_ref[...] = jnp.zeros((1, 1), jnp.float32)

    out_ref[...] += part


_tc_loss = pl.pallas_call(
    _tc_loss_body,
    grid=(B // BLK,),
    in_specs=[
        pl.BlockSpec((BLK, EMB), lambda i: (i, 0)),
        pl.BlockSpec((B, EMB), lambda i: (0, 0)),
        pl.BlockSpec((BLK, L), lambda i: (i, 0)),
        pl.BlockSpec((B, L), lambda i: (0, 0)),
        pl.BlockSpec((1, EMB), lambda i: (0, 0)),
    ],
    out_specs=pl.BlockSpec((1, 1), lambda i: (0, 0)),
    out_shape=jax.ShapeDtypeStruct((1, 1), jnp.float32),
    compiler_params=pltpu.CompilerParams(
        dimension_semantics=("arbitrary",),
    ),
)


def kernel(sentences1, sentences2, emb_table):
    sents = jnp.concatenate([sentences1, sentences2], axis=0).reshape(
        NW, IDX_ROWS_PER_W, IDX_COLS
    )
    pooled = _sc_pool(sents, emb_table)
    row0 = lax.slice(emb_table, (0, 0), (1, EMB))
    loss = _tc_loss(pooled[:B], pooled[B:], sentences1, sentences2, row0)
    return loss[0, 0]
